# baseline (speedup 1.0000x reference)
import functools

import jax
import jax.numpy as jnp
from jax import lax
from jax.experimental import pallas as pl
from jax.experimental.pallas import tpu as pltpu
from jax.experimental.pallas import tpu_sc as plsc

_LANES = 16
_CHUNK = 8


def kernel(inputs, relative_embedding):
    seq_len = inputs.shape[1]
    num_rows, d = relative_embedding.shape
    max_d = (num_rows - 1) // 2
    lo, hi = -seq_len + 1, seq_len - 1

    n_pad = ((num_rows + _CHUNK - 1) // _CHUNK) * _CHUNK
    n_workers = n_pad // _CHUNK

    mesh = plsc.VectorSubcoreMesh(
        core_axis_name="c", subcore_axis_name="s", num_cores=1
    )

    @functools.partial(
        pl.kernel,
        mesh=mesh,
        out_type=jax.ShapeDtypeStruct((n_pad, d), jnp.float32),
        scratch_types=[
            pltpu.VMEM((_LANES,), jnp.int32),
            pltpu.VMEM((_CHUNK, d), jnp.float32),
            pltpu.SemaphoreType.DMA,
        ],
    )
    def emb_gather(table_hbm, out_hbm, idx_v, rows_v, sem):
        wid = lax.axis_index("s")
        base = wid * _CHUNK

        p = lax.iota(jnp.int32, _LANES) + base
        r = jnp.minimum(jnp.maximum(p - max_d, lo), hi) + max_d
        idx_v[...] = jnp.minimum(r, num_rows - 1)

        @pl.when(wid < n_workers)
        def _():
            pltpu.async_copy(
                table_hbm.at[idx_v.at[pl.ds(0, _CHUNK)]], rows_v, sem
            ).wait()
            pltpu.sync_copy(rows_v, out_hbm.at[pl.ds(base, _CHUNK)])

    return emb_gather(relative_embedding)[:num_rows]


# speedup vs baseline: 1.0114x; 1.0114x over previous
import functools

import jax
import jax.numpy as jnp
from jax import lax
from jax.experimental import pallas as pl
from jax.experimental.pallas import tpu as pltpu
from jax.experimental.pallas import tpu_sc as plsc

_LANES = 16
_CHUNK = 8


def kernel(inputs, relative_embedding):
    seq_len = inputs.shape[1]
    num_rows, d = relative_embedding.shape
    max_d = (num_rows - 1) // 2
    lo, hi = -seq_len + 1, seq_len - 1

    n_full = num_rows // _CHUNK
    rem = num_rows - n_full * _CHUNK

    mesh = plsc.VectorSubcoreMesh(
        core_axis_name="c", subcore_axis_name="s", num_cores=1
    )

    @functools.partial(
        pl.kernel,
        mesh=mesh,
        out_type=jax.ShapeDtypeStruct((num_rows, d), jnp.float32),
        scratch_types=[
            pltpu.VMEM((_LANES,), jnp.int32),
            pltpu.VMEM((_CHUNK, d), jnp.float32),
            pltpu.SemaphoreType.DMA,
        ],
    )
    def emb_gather(table_hbm, out_hbm, idx_v, rows_v, sem):
        wid = lax.axis_index("s")
        base = wid * _CHUNK

        p = lax.iota(jnp.int32, _LANES) + base
        r = jnp.minimum(jnp.maximum(p - max_d, lo), hi) + max_d
        idx_v[...] = jnp.minimum(r, num_rows - 1)

        @pl.when(wid < n_full)
        def _full():
            pltpu.async_copy(
                table_hbm.at[idx_v.at[pl.ds(0, _CHUNK)]], rows_v, sem
            ).wait()
            pltpu.sync_copy(rows_v, out_hbm.at[pl.ds(base, _CHUNK)])

        if rem:

            @pl.when(wid == n_full)
            def _tail():
                pltpu.async_copy(
                    table_hbm.at[idx_v.at[pl.ds(0, _CHUNK)]], rows_v, sem
                ).wait()
                pltpu.sync_copy(
                    rows_v.at[pl.ds(0, rem)], out_hbm.at[pl.ds(base, rem)]
                )

    return emb_gather(relative_embedding)


# speedup vs baseline: 1.0347x; 1.0230x over previous
import functools

import jax
import jax.numpy as jnp
from jax import lax
from jax.experimental import pallas as pl
from jax.experimental.pallas import tpu as pltpu
from jax.experimental.pallas import tpu_sc as plsc

_LANES = 16
_CHUNK = 8


def kernel(inputs, relative_embedding):
    seq_len = inputs.shape[1]
    num_rows, d = relative_embedding.shape
    max_d = (num_rows - 1) // 2
    lo, hi = -seq_len + 1, seq_len - 1

    n_full = num_rows // _CHUNK
    rem = num_rows - n_full * _CHUNK

    mesh = plsc.VectorSubcoreMesh(
        core_axis_name="c", subcore_axis_name="s", num_cores=1
    )

    n_row_chunks = n_full + (1 if rem else 0)
    dh = d // 2

    @functools.partial(
        pl.kernel,
        mesh=mesh,
        out_type=jax.ShapeDtypeStruct((num_rows, d), jnp.float32),
        scratch_types=[
            pltpu.VMEM((_LANES,), jnp.int32),
            pltpu.VMEM((_CHUNK, dh), jnp.float32),
            pltpu.SemaphoreType.DMA,
        ],
    )
    def emb_gather(table_hbm, out_hbm, idx_v, rows_v, sem):
        wid = lax.axis_index("s")
        rc = wid // 2
        coff = (wid % 2) * dh
        base = rc * _CHUNK

        p = lax.iota(jnp.int32, _LANES) + base
        r = jnp.minimum(jnp.maximum(p - max_d, lo), hi) + max_d
        idx_v[...] = jnp.minimum(r, num_rows - 1)

        @pl.when(rc < n_full)
        def _full():
            pltpu.async_copy(
                table_hbm.at[idx_v.at[pl.ds(0, _CHUNK)], pl.ds(coff, dh)],
                rows_v,
                sem,
            ).wait()
            pltpu.sync_copy(
                rows_v, out_hbm.at[pl.ds(base, _CHUNK), pl.ds(coff, dh)]
            )

        if rem:

            @pl.when(rc == n_full)
            def _tail():
                pltpu.async_copy(
                    table_hbm.at[idx_v.at[pl.ds(0, _CHUNK)], pl.ds(coff, dh)],
                    rows_v,
                    sem,
                ).wait()
                pltpu.sync_copy(
                    rows_v.at[pl.ds(0, rem)],
                    out_hbm.at[pl.ds(base, rem), pl.ds(coff, dh)],
                )

    return emb_gather(relative_embedding)
